# baseline (speedup 1.0000x reference)
import functools

import jax
import jax.numpy as jnp
from jax import lax
from jax.experimental import pallas as pl
from jax.experimental.pallas import tpu as pltpu
from jax.experimental.pallas import tpu_sc as plsc

_N = 131072
_D = _H = _W = 128
_NC = 2
_NS = 16
_CHUNK = 2048
_PER_TILE = _N // _NS
_SUBCHUNKS = _PER_TILE // _CHUNK
_HALF_WORDS = (_D // _NC) * _H * _W
_TILE_WORDS = _HALF_WORDS // _NS
_GROUPS = _CHUNK // 16
_ZCHUNK = 16384


def _floor_i32(f):
    t = f.astype(jnp.int32)
    tf = t.astype(jnp.float32)
    return jnp.where(tf > f, t - 1, t)


def _axis_terms(coord, lim):
    f = ((coord + 1.0) * 128.0 - 1.0) * 0.5
    c0 = _floor_i32(f)
    t = f - c0.astype(jnp.float32)
    v0 = (c0 >= 0) & (c0 <= lim - 1)
    v1 = (c0 >= -1) & (c0 <= lim - 2)
    a0 = jnp.where(v0, 1.0 - t, 0.0)
    a1 = jnp.where(v1, t, 0.0)
    i0 = jnp.clip(c0, 0, lim - 1)
    i1 = jnp.clip(c0 + 1, 0, lim - 1)
    return a0, a1, i0, i1


def _sc_body(px_hbm, py_hbm, pz_hbm, gx_hbm, gy_hbm, gz_hbm,
             cx_hbm, cy_hbm, cz_hbm, zeros_hbm, out_hbm,
             pxb, pyb, pzb, cxb, cyb, czb, idx_st, val_st, zb, accv, grid,
             in_sem, scat_sem):
    pb = (pxb, pyb, pzb)
    cb = (cxb, cyb, czb)
    pred_hbm = (px_hbm, py_hbm, pz_hbm)
    gt_hbm = (gx_hbm, gy_hbm, gz_hbm)
    coords_hbm = (cx_hbm, cy_hbm, cz_hbm)
    ci = lax.axis_index("c")
    si = lax.axis_index("s")
    zoff = ci * (_D // _NC)
    tile_base = si * _TILE_WORDS

    zeros16 = jnp.zeros((16,), jnp.float32)

    pltpu.sync_copy(zeros_hbm.at[pl.ds(si * _TILE_WORDS, _TILE_WORDS)],
                    grid.at[pl.ds(tile_base, _TILE_WORDS)])
    plsc.subcore_barrier()

    for cloud, (src_hbm, sign) in enumerate(((pred_hbm, 1.0), (gt_hbm, -1.0))):
        def _chunk_body(sub, carry, src_hbm=src_hbm, sign=sign):
            base = si * _PER_TILE + sub * _CHUNK
            for d in range(3):
                pltpu.async_copy(src_hbm[d].at[pl.ds(base, _CHUNK)], pb[d],
                                 in_sem)
                pltpu.async_copy(coords_hbm[d].at[pl.ds(base, _CHUNK)], cb[d],
                                 in_sem)
            pltpu.make_async_copy(px_hbm.at[pl.ds(0, 6 * _CHUNK)],
                                  zb.at[pl.ds(0, 6 * _CHUNK)], in_sem).wait()

            def _group(i, c2):
                s16 = pl.ds(i * 16, 16)
                x = pb[0][s16] + cb[0][s16]
                y = pb[1][s16] + cb[1][s16]
                z = pb[2][s16] + cb[2][s16]
                ax0, ax1, xi0, xi1 = _axis_terms(x, _W)
                ay0, ay1, yi0, yi1 = _axis_terms(y, _H)
                fz = ((z + 1.0) * 128.0 - 1.0) * 0.5
                z0 = _floor_i32(fz)
                tz = fz - z0.astype(jnp.float32)
                z0l = z0 - zoff
                vz0 = (z0l >= 0) & (z0l <= 63)
                vz1 = (z0l >= -1) & (z0l <= 62)
                az0 = jnp.where(vz0, (1.0 - tz) * sign, 0.0)
                az1 = jnp.where(vz1, tz * sign, 0.0)
                zb0 = jnp.clip(z0l, 0, 63) * (_H * _W)
                zb1 = jnp.clip(z0l + 1, 0, 63) * (_H * _W)

                yb0 = yi0 * _W
                yb1 = yi1 * _W
                b00 = zb0 + yb0
                b01 = zb0 + yb1
                b10 = zb1 + yb0
                b11 = zb1 + yb1
                a00 = az0 * ay0
                a01 = az0 * ay1
                a10 = az1 * ay0
                a11 = az1 * ay1
                corners = (
                    (b00, a00), (b01, a01), (b10, a10), (b11, a11))
                for k2, (bzy, azy) in enumerate(corners):
                    idx_st[i, pl.ds((2 * k2) * 16, 16)] = bzy + xi0
                    val_st[i, pl.ds((2 * k2) * 16, 16)] = azy * ax0
                    idx_st[i, pl.ds((2 * k2 + 1) * 16, 16)] = bzy + xi1
                    val_st[i, pl.ds((2 * k2 + 1) * 16, 16)] = azy * ax1
                return c2
            lax.fori_loop(0, _GROUPS, _group, 0)

            def _scat(j, c3):
                pltpu.async_copy(val_st.at[j], grid.at[idx_st.at[j]],
                                 scat_sem, add=True)
                return c3
            lax.fori_loop(0, _GROUPS, _scat, 0)
            pltpu.make_async_copy(px_hbm.at[pl.ds(0, _ZCHUNK)], zb,
                                  scat_sem).wait()
            return carry
        lax.fori_loop(0, _SUBCHUNKS, _chunk_body, 0)

    plsc.subcore_barrier()

    pltpu.sync_copy(grid.at[pl.ds(tile_base, _TILE_WORDS)],
                    out_hbm.at[pl.ds(ci * _HALF_WORDS + tile_base,
                                     _TILE_WORDS)])


def _huber_body(g_ref, o_ref):
    d = g_ref[...]
    ad = jnp.abs(d)
    o_ref[0, 0] = jnp.sum(jnp.where(ad < 1.0, 0.5 * d * d, ad - 0.5))


@jax.jit
def _divroc_sc(px, py, pz, gx, gy, gz, cx, cy, cz):
    mesh = plsc.VectorSubcoreMesh(
        core_axis_name="c", subcore_axis_name="s",
        num_cores=_NC, num_subcores=_NS)
    fn = pl.kernel(
        _sc_body,
        out_type=jax.ShapeDtypeStruct((_NC * _HALF_WORDS,), jnp.float32),
        mesh=mesh,
        scratch_types=[
            pltpu.VMEM((_CHUNK,), jnp.float32),
            pltpu.VMEM((_CHUNK,), jnp.float32),
            pltpu.VMEM((_CHUNK,), jnp.float32),
            pltpu.VMEM((_CHUNK,), jnp.float32),
            pltpu.VMEM((_CHUNK,), jnp.float32),
            pltpu.VMEM((_CHUNK,), jnp.float32),
            pltpu.VMEM((_GROUPS, 128), jnp.int32),
            pltpu.VMEM((_GROUPS, 128), jnp.float32),
            pltpu.VMEM((_ZCHUNK,), jnp.float32),
            pltpu.VMEM((16,), jnp.float32),
            pltpu.VMEM_SHARED((_HALF_WORDS,), jnp.float32),
            pltpu.SemaphoreType.DMA,
            pltpu.SemaphoreType.DMA,
        ],
    )
    zeros_seed = jnp.zeros((_HALF_WORDS,), jnp.float32)
    diff_grid = fn(px, py, pz, gx, gy, gz, cx, cy, cz, zeros_seed)
    hsum = pl.pallas_call(
        _huber_body,
        out_shape=jax.ShapeDtypeStruct((1, 1), jnp.float32),
        out_specs=pl.BlockSpec(memory_space=pltpu.SMEM),
    )(diff_grid.reshape(1024, 2048))
    return hsum[0, 0]


def kernel(registration_pred, registration_gt, coords, wandb):
    n = registration_pred.shape[1]
    p = registration_pred.reshape(n, 3)
    g = registration_gt.reshape(n, 3)
    c = coords.reshape(n, 3)
    return _divroc_sc(p[:, 0], p[:, 1], p[:, 2],
                      g[:, 0], g[:, 1], g[:, 2],
                      c[:, 0], c[:, 1], c[:, 2])


# speedup vs baseline: 1.6780x; 1.6780x over previous
import jax
import jax.numpy as jnp
from jax import lax
from jax.experimental import pallas as pl
from jax.experimental.pallas import tpu as pltpu
from jax.experimental.pallas import tpu_sc as plsc

_N = 131072
_D = _H = _W = 128
_NC = 2
_NS = 16
_CHUNK = 2048
_PER_TILE = _N // _NS
_SUBCHUNKS = _PER_TILE // _CHUNK
_HALF_WORDS = (_D // _NC) * _H * _W
_TILE_WORDS = _HALF_WORDS // _NS
_GROUPS = _CHUNK // 16
_ROWS = _CHUNK // 32
_ZCHUNK = 16384


def _floor_i32(f):
    t = f.astype(jnp.int32)
    tf = t.astype(jnp.float32)
    return jnp.where(tf > f, t - 1, t)


def _axis_terms(coord, lim):
    f = ((coord + 1.0) * 128.0 - 1.0) * 0.5
    c0 = _floor_i32(f)
    t = f - c0.astype(jnp.float32)
    v0 = (c0 >= 0) & (c0 <= lim - 1)
    v1 = (c0 >= -1) & (c0 <= lim - 2)
    a0 = jnp.where(v0, 1.0 - t, 0.0)
    a1 = jnp.where(v1, t, 0.0)
    i0 = jnp.clip(c0, 0, lim - 1)
    i1 = jnp.clip(c0 + 1, 0, lim - 1)
    return a0, a1, i0, i1


def _sc_body(px_hbm, py_hbm, pz_hbm, gx_hbm, gy_hbm, gz_hbm,
             cx_hbm, cy_hbm, cz_hbm, zeros_hbm, out_hbm,
             pxb, pyb, pzb, cxb, cyb, czb,
             idx_st0, val_st0, idx_st1, val_st1, zb, grid,
             in_sem, scat_sem):
    pb = (pxb, pyb, pzb)
    cb = (cxb, cyb, czb)
    pred_hbm = (px_hbm, py_hbm, pz_hbm)
    gt_hbm = (gx_hbm, gy_hbm, gz_hbm)
    coords_hbm = (cx_hbm, cy_hbm, cz_hbm)
    stages = ((idx_st0, val_st0), (idx_st1, val_st1))
    ci = lax.axis_index("c")
    si = lax.axis_index("s")
    tile_base = si * _TILE_WORDS

    pltpu.sync_copy(zeros_hbm.at[pl.ds(si * _TILE_WORDS, _TILE_WORDS)],
                    grid.at[pl.ds(tile_base, _TILE_WORDS)])
    plsc.subcore_barrier()

    phase = 0
    for src_hbm, sign in ((pred_hbm, 1.0), (gt_hbm, -1.0)):
        for sub in range(_SUBCHUNKS):
            ist, vst = stages[phase % 2]
            base = si * _PER_TILE + sub * _CHUNK
            for d in range(3):
                pltpu.async_copy(src_hbm[d].at[pl.ds(base, _CHUNK)], pb[d],
                                 in_sem)
                pltpu.async_copy(coords_hbm[d].at[pl.ds(base, _CHUNK)], cb[d],
                                 in_sem)
            pltpu.make_async_copy(px_hbm.at[pl.ds(0, 6 * _CHUNK)],
                                  zb.at[pl.ds(0, 6 * _CHUNK)], in_sem).wait()

            def _group(i, c2, ist=ist, vst=vst, sign=sign):
                s16 = pl.ds(i * 16, 16)
                x = pb[0][s16] + cb[0][s16]
                y = pb[1][s16] + cb[1][s16]
                z = pb[2][s16] + cb[2][s16]
                ax0, ax1, xi0, xi1 = _axis_terms(x, _W)
                ay0, ay1, yi0, yi1 = _axis_terms(y, _H)
                fz = ((z + 1.0) * 128.0 - 1.0) * 0.5
                z0 = _floor_i32(fz)
                tz = fz - z0.astype(jnp.float32)
                dz = (z0 & 1) ^ ci
                zc = z0 + dz
                wz = jnp.where(dz == 0, 1.0 - tz, tz)
                vzc = (zc >= 0) & (zc <= _D - 1)
                azc = jnp.where(vzc, wz * sign, 0.0)
                zbase = (jnp.clip(zc, 0, _D - 1) >> 1) * (_H * _W)

                b0 = zbase + yi0 * _W
                b1 = zbase + yi1 * _W
                a0 = azc * ay0
                a1 = azc * ay1
                row = i >> 1
                colb = (i & 1) * 64
                ist[row, pl.ds(colb, 16)] = b0 + xi0
                vst[row, pl.ds(colb, 16)] = a0 * ax0
                ist[row, pl.ds(colb + 16, 16)] = b0 + xi1
                vst[row, pl.ds(colb + 16, 16)] = a0 * ax1
                ist[row, pl.ds(colb + 32, 16)] = b1 + xi0
                vst[row, pl.ds(colb + 32, 16)] = a1 * ax0
                ist[row, pl.ds(colb + 48, 16)] = b1 + xi1
                vst[row, pl.ds(colb + 48, 16)] = a1 * ax1
                return c2
            lax.fori_loop(0, _GROUPS, _group, 0)

            if phase > 0:
                pltpu.make_async_copy(px_hbm.at[pl.ds(0, _ROWS * 128)],
                                      zb.at[pl.ds(0, _ROWS * 128)],
                                      scat_sem).wait()

            def _scat(j, c3, ist=ist, vst=vst):
                pltpu.async_copy(vst.at[j], grid.at[ist.at[j]],
                                 scat_sem, add=True)
                return c3
            lax.fori_loop(0, _ROWS, _scat, 0)
            phase += 1

    pltpu.make_async_copy(px_hbm.at[pl.ds(0, _ROWS * 128)],
                          zb.at[pl.ds(0, _ROWS * 128)], scat_sem).wait()
    plsc.subcore_barrier()

    pltpu.sync_copy(grid.at[pl.ds(tile_base, _TILE_WORDS)],
                    out_hbm.at[pl.ds(ci * _HALF_WORDS + tile_base,
                                     _TILE_WORDS)])


def _huber_body(g_ref, o_ref):
    d = g_ref[...]
    ad = jnp.abs(d)
    o_ref[0, 0] = jnp.sum(jnp.where(ad < 1.0, 0.5 * d * d, ad - 0.5))


@jax.jit
def _divroc_sc(px, py, pz, gx, gy, gz, cx, cy, cz):
    mesh = plsc.VectorSubcoreMesh(
        core_axis_name="c", subcore_axis_name="s",
        num_cores=_NC, num_subcores=_NS)
    fn = pl.kernel(
        _sc_body,
        out_type=jax.ShapeDtypeStruct((_NC * _HALF_WORDS,), jnp.float32),
        mesh=mesh,
        scratch_types=[
            pltpu.VMEM((_CHUNK,), jnp.float32),
            pltpu.VMEM((_CHUNK,), jnp.float32),
            pltpu.VMEM((_CHUNK,), jnp.float32),
            pltpu.VMEM((_CHUNK,), jnp.float32),
            pltpu.VMEM((_CHUNK,), jnp.float32),
            pltpu.VMEM((_CHUNK,), jnp.float32),
            pltpu.VMEM((_ROWS, 128), jnp.int32),
            pltpu.VMEM((_ROWS, 128), jnp.float32),
            pltpu.VMEM((_ROWS, 128), jnp.int32),
            pltpu.VMEM((_ROWS, 128), jnp.float32),
            pltpu.VMEM((_ZCHUNK,), jnp.float32),
            pltpu.VMEM_SHARED((_HALF_WORDS,), jnp.float32),
            pltpu.SemaphoreType.DMA,
            pltpu.SemaphoreType.DMA,
        ],
    )
    zeros_seed = jnp.zeros((_HALF_WORDS,), jnp.float32)
    diff_grid = fn(px, py, pz, gx, gy, gz, cx, cy, cz, zeros_seed)
    hsum = pl.pallas_call(
        _huber_body,
        out_shape=jax.ShapeDtypeStruct((1, 1), jnp.float32),
        out_specs=pl.BlockSpec(memory_space=pltpu.SMEM),
    )(diff_grid.reshape(1024, 2048))
    return hsum[0, 0]


def kernel(registration_pred, registration_gt, coords, wandb):
    n = registration_pred.shape[1]
    p = registration_pred.reshape(n, 3)
    g = registration_gt.reshape(n, 3)
    c = coords.reshape(n, 3)
    return _divroc_sc(p[:, 0], p[:, 1], p[:, 2],
                      g[:, 0], g[:, 1], g[:, 2],
                      c[:, 0], c[:, 1], c[:, 2])


# speedup vs baseline: 1.7292x; 1.0305x over previous
import jax
import jax.numpy as jnp
from jax import lax
from jax.experimental import pallas as pl
from jax.experimental.pallas import tpu as pltpu
from jax.experimental.pallas import tpu_sc as plsc

_N = 131072
_D = _H = _W = 128
_NC = 2
_NS = 16
_CHUNK = 2048
_PER_TILE = _N // _NS
_SUBCHUNKS = _PER_TILE // _CHUNK
_HALF_WORDS = (_D // _NC) * _H * _W
_TILE_WORDS = _HALF_WORDS // _NS
_GROUPS = _CHUNK // 16
_ROWS = _CHUNK // 32
_ZCHUNK = 16384


def _floor_i32(f):
    t = f.astype(jnp.int32)
    tf = t.astype(jnp.float32)
    return jnp.where(tf > f, t - 1, t)


def _axis_terms(coord, lim):
    f = ((coord + 1.0) * 128.0 - 1.0) * 0.5
    c0 = _floor_i32(f)
    t = f - c0.astype(jnp.float32)
    v0 = (c0 >= 0) & (c0 <= lim - 1)
    v1 = (c0 >= -1) & (c0 <= lim - 2)
    a0 = jnp.where(v0, 1.0 - t, 0.0)
    a1 = jnp.where(v1, t, 0.0)
    i0 = jnp.clip(c0, 0, lim - 1)
    i1 = jnp.clip(c0 + 1, 0, lim - 1)
    return a0, a1, i0, i1


def _sc_body(px_hbm, py_hbm, pz_hbm, gx_hbm, gy_hbm, gz_hbm,
             cx_hbm, cy_hbm, cz_hbm, zeros_hbm, out_hbm,
             pxb0, pyb0, pzb0, cxb0, cyb0, czb0,
             pxb1, pyb1, pzb1, cxb1, cyb1, czb1,
             idx_st0, val_st0, idx_st1, val_st1, zb, grid,
             in_sem, scat_sem):
    inbufs = (((pxb0, pyb0, pzb0), (cxb0, cyb0, czb0)),
              ((pxb1, pyb1, pzb1), (cxb1, cyb1, czb1)))
    pred_hbm = (px_hbm, py_hbm, pz_hbm)
    gt_hbm = (gx_hbm, gy_hbm, gz_hbm)
    coords_hbm = (cx_hbm, cy_hbm, cz_hbm)
    stages = ((idx_st0, val_st0), (idx_st1, val_st1))
    ci = lax.axis_index("c")
    si = lax.axis_index("s")
    tile_base = si * _TILE_WORDS

    pltpu.sync_copy(zeros_hbm.at[pl.ds(si * _TILE_WORDS, _TILE_WORDS)],
                    grid.at[pl.ds(tile_base, _TILE_WORDS)])
    plsc.subcore_barrier()

    def _fire_loads(ph):
        src_hbm = pred_hbm if ph < _SUBCHUNKS else gt_hbm
        base = si * _PER_TILE + (ph % _SUBCHUNKS) * _CHUNK
        pbs, cbs = inbufs[ph % 2]
        for d in range(3):
            pltpu.async_copy(src_hbm[d].at[pl.ds(base, _CHUNK)], pbs[d],
                             in_sem)
            pltpu.async_copy(coords_hbm[d].at[pl.ds(base, _CHUNK)], cbs[d],
                             in_sem)

    _fire_loads(0)
    phase = 0
    for src_hbm, sign in ((pred_hbm, 1.0), (gt_hbm, -1.0)):
        for sub in range(_SUBCHUNKS):
            ist, vst = stages[phase % 2]
            pb, cb = inbufs[phase % 2]
            pltpu.make_async_copy(px_hbm.at[pl.ds(0, 6 * _CHUNK)],
                                  zb.at[pl.ds(0, 6 * _CHUNK)], in_sem).wait()
            if phase + 1 < 2 * _SUBCHUNKS:
                _fire_loads(phase + 1)

            def _group(i, c2, ist=ist, vst=vst, sign=sign, pb=pb, cb=cb):
                s16 = pl.ds(i * 16, 16)
                x = pb[0][s16] + cb[0][s16]
                y = pb[1][s16] + cb[1][s16]
                z = pb[2][s16] + cb[2][s16]
                ax0, ax1, xi0, xi1 = _axis_terms(x, _W)
                ay0, ay1, yi0, yi1 = _axis_terms(y, _H)
                fz = ((z + 1.0) * 128.0 - 1.0) * 0.5
                z0 = _floor_i32(fz)
                tz = fz - z0.astype(jnp.float32)
                dz = (z0 & 1) ^ ci
                zc = z0 + dz
                wz = jnp.where(dz == 0, 1.0 - tz, tz)
                vzc = (zc >= 0) & (zc <= _D - 1)
                azc = jnp.where(vzc, wz * sign, 0.0)
                zbase = (jnp.clip(zc, 0, _D - 1) >> 1) * (_H * _W)

                b0 = zbase + yi0 * _W
                b1 = zbase + yi1 * _W
                a0 = azc * ay0
                a1 = azc * ay1
                row = i >> 1
                colb = (i & 1) * 64
                ist[row, pl.ds(colb, 16)] = b0 + xi0
                vst[row, pl.ds(colb, 16)] = a0 * ax0
                ist[row, pl.ds(colb + 16, 16)] = b0 + xi1
                vst[row, pl.ds(colb + 16, 16)] = a0 * ax1
                ist[row, pl.ds(colb + 32, 16)] = b1 + xi0
                vst[row, pl.ds(colb + 32, 16)] = a1 * ax0
                ist[row, pl.ds(colb + 48, 16)] = b1 + xi1
                vst[row, pl.ds(colb + 48, 16)] = a1 * ax1
                return c2
            lax.fori_loop(0, _GROUPS, _group, 0, unroll=2)

            if phase > 0:
                pltpu.make_async_copy(px_hbm.at[pl.ds(0, _ROWS * 128)],
                                      zb.at[pl.ds(0, _ROWS * 128)],
                                      scat_sem).wait()

            def _scat(j, c3, ist=ist, vst=vst):
                pltpu.async_copy(vst.at[j], grid.at[ist.at[j]],
                                 scat_sem, add=True)
                return c3
            lax.fori_loop(0, _ROWS, _scat, 0)
            phase += 1

    pltpu.make_async_copy(px_hbm.at[pl.ds(0, _ROWS * 128)],
                          zb.at[pl.ds(0, _ROWS * 128)], scat_sem).wait()
    plsc.subcore_barrier()

    pltpu.sync_copy(grid.at[pl.ds(tile_base, _TILE_WORDS)],
                    out_hbm.at[pl.ds(ci * _HALF_WORDS + tile_base,
                                     _TILE_WORDS)])


def _huber_body(g_ref, o_ref):
    d = g_ref[...]
    ad = jnp.abs(d)
    o_ref[0, 0] = jnp.sum(jnp.where(ad < 1.0, 0.5 * d * d, ad - 0.5))


@jax.jit
def _divroc_sc(px, py, pz, gx, gy, gz, cx, cy, cz):
    mesh = plsc.VectorSubcoreMesh(
        core_axis_name="c", subcore_axis_name="s",
        num_cores=_NC, num_subcores=_NS)
    fn = pl.kernel(
        _sc_body,
        out_type=jax.ShapeDtypeStruct((_NC * _HALF_WORDS,), jnp.float32),
        mesh=mesh,
        scratch_types=[
            pltpu.VMEM((_CHUNK,), jnp.float32),
            pltpu.VMEM((_CHUNK,), jnp.float32),
            pltpu.VMEM((_CHUNK,), jnp.float32),
            pltpu.VMEM((_CHUNK,), jnp.float32),
            pltpu.VMEM((_CHUNK,), jnp.float32),
            pltpu.VMEM((_CHUNK,), jnp.float32),
            pltpu.VMEM((_CHUNK,), jnp.float32),
            pltpu.VMEM((_CHUNK,), jnp.float32),
            pltpu.VMEM((_CHUNK,), jnp.float32),
            pltpu.VMEM((_CHUNK,), jnp.float32),
            pltpu.VMEM((_CHUNK,), jnp.float32),
            pltpu.VMEM((_CHUNK,), jnp.float32),
            pltpu.VMEM((_ROWS, 128), jnp.int32),
            pltpu.VMEM((_ROWS, 128), jnp.float32),
            pltpu.VMEM((_ROWS, 128), jnp.int32),
            pltpu.VMEM((_ROWS, 128), jnp.float32),
            pltpu.VMEM((_ZCHUNK,), jnp.float32),
            pltpu.VMEM_SHARED((_HALF_WORDS,), jnp.float32),
            pltpu.SemaphoreType.DMA,
            pltpu.SemaphoreType.DMA,
        ],
    )
    zeros_seed = jnp.zeros((_HALF_WORDS,), jnp.float32)
    diff_grid = fn(px, py, pz, gx, gy, gz, cx, cy, cz, zeros_seed)
    hsum = pl.pallas_call(
        _huber_body,
        out_shape=jax.ShapeDtypeStruct((1, 1), jnp.float32),
        out_specs=pl.BlockSpec(memory_space=pltpu.SMEM),
    )(diff_grid.reshape(1024, 2048))
    return hsum[0, 0]


def kernel(registration_pred, registration_gt, coords, wandb):
    n = registration_pred.shape[1]
    p = registration_pred.reshape(n, 3)
    g = registration_gt.reshape(n, 3)
    c = coords.reshape(n, 3)
    return _divroc_sc(p[:, 0], p[:, 1], p[:, 2],
                      g[:, 0], g[:, 1], g[:, 2],
                      c[:, 0], c[:, 1], c[:, 2])


# speedup vs baseline: 1.7385x; 1.0053x over previous
import jax
import jax.numpy as jnp
from jax import lax
from jax.experimental import pallas as pl
from jax.experimental.pallas import tpu as pltpu
from jax.experimental.pallas import tpu_sc as plsc

_N = 131072
_D = _H = _W = 128
_NC = 2
_NS = 16
_CHUNK = 2048
_PER_TILE = _N // _NS
_SUBCHUNKS = _PER_TILE // _CHUNK
_HALF_WORDS = (_D // _NC) * _H * _W
_TILE_WORDS = _HALF_WORDS // _NS
_GROUPS = _CHUNK // 16
_ROWS = _CHUNK // 32
_ZCHUNK = 16384


def _floor_i32(f):
    t = f.astype(jnp.int32)
    tf = t.astype(jnp.float32)
    return jnp.where(tf > f, t - 1, t)


def _axis_terms(coord, lim):
    f = ((coord + 1.0) * 128.0 - 1.0) * 0.5
    c0 = _floor_i32(f)
    t = f - c0.astype(jnp.float32)
    v0 = (c0 >= 0) & (c0 <= lim - 1)
    v1 = (c0 >= -1) & (c0 <= lim - 2)
    a0 = jnp.where(v0, 1.0 - t, 0.0)
    a1 = jnp.where(v1, t, 0.0)
    i0 = jnp.clip(c0, 0, lim - 1)
    i1 = jnp.clip(c0 + 1, 0, lim - 1)
    return a0, a1, i0, i1


def _sc_body(px_hbm, py_hbm, pz_hbm, gx_hbm, gy_hbm, gz_hbm,
             cx_hbm, cy_hbm, cz_hbm, zeros_hbm, out_hbm,
             pxb, pyb, pzb, cxb, cyb, czb,
             idx_st0, val_st0, idx_st1, val_st1, zb, grid,
             in_sem, scat_sem):
    pb = (pxb, pyb, pzb)
    cb = (cxb, cyb, czb)
    pred_hbm = (px_hbm, py_hbm, pz_hbm)
    gt_hbm = (gx_hbm, gy_hbm, gz_hbm)
    coords_hbm = (cx_hbm, cy_hbm, cz_hbm)
    stages = ((idx_st0, val_st0), (idx_st1, val_st1))
    ci = lax.axis_index("c")
    si = lax.axis_index("s")
    tile_base = si * _TILE_WORDS

    pltpu.sync_copy(zeros_hbm.at[pl.ds(si * _TILE_WORDS, _TILE_WORDS)],
                    grid.at[pl.ds(tile_base, _TILE_WORDS)])
    plsc.subcore_barrier()

    phase = 0
    for src_hbm, sign in ((pred_hbm, 1.0), (gt_hbm, -1.0)):
        for sub in range(_SUBCHUNKS):
            ist, vst = stages[phase % 2]
            base = si * _PER_TILE + sub * _CHUNK
            for d in range(3):
                pltpu.async_copy(src_hbm[d].at[pl.ds(base, _CHUNK)], pb[d],
                                 in_sem)
                pltpu.async_copy(coords_hbm[d].at[pl.ds(base, _CHUNK)], cb[d],
                                 in_sem)
            pltpu.make_async_copy(px_hbm.at[pl.ds(0, 6 * _CHUNK)],
                                  zb.at[pl.ds(0, 6 * _CHUNK)], in_sem).wait()

            def _group(i, c2, ist=ist, vst=vst, sign=sign):
                s16 = pl.ds(i * 16, 16)
                x = pb[0][s16] + cb[0][s16]
                y = pb[1][s16] + cb[1][s16]
                z = pb[2][s16] + cb[2][s16]
                ax0, ax1, xi0, xi1 = _axis_terms(x, _W)
                ay0, ay1, yi0, yi1 = _axis_terms(y, _H)
                fz = ((z + 1.0) * 128.0 - 1.0) * 0.5
                z0 = _floor_i32(fz)
                tz = fz - z0.astype(jnp.float32)
                dz = (z0 & 1) ^ ci
                zc = z0 + dz
                wz = jnp.where(dz == 0, 1.0 - tz, tz)
                vzc = (zc >= 0) & (zc <= _D - 1)
                azc = jnp.where(vzc, wz * sign, 0.0)
                zbase = (jnp.clip(zc, 0, _D - 1) >> 1) * (_H * _W)

                b0 = zbase + yi0 * _W
                b1 = zbase + yi1 * _W
                a0 = azc * ay0
                a1 = azc * ay1
                row = i >> 1
                colb = (i & 1) * 64
                ist[row, pl.ds(colb, 16)] = b0 + xi0
                vst[row, pl.ds(colb, 16)] = a0 * ax0
                ist[row, pl.ds(colb + 16, 16)] = b0 + xi1
                vst[row, pl.ds(colb + 16, 16)] = a0 * ax1
                ist[row, pl.ds(colb + 32, 16)] = b1 + xi0
                vst[row, pl.ds(colb + 32, 16)] = a1 * ax0
                ist[row, pl.ds(colb + 48, 16)] = b1 + xi1
                vst[row, pl.ds(colb + 48, 16)] = a1 * ax1
                return c2
            lax.fori_loop(0, _GROUPS, _group, 0)

            if phase > 0:
                pltpu.make_async_copy(px_hbm.at[pl.ds(0, _ROWS * 128)],
                                      zb.at[pl.ds(0, _ROWS * 128)],
                                      scat_sem).wait()

            def _scat(j, c3, ist=ist, vst=vst):
                pltpu.async_copy(vst.at[j], grid.at[ist.at[j]],
                                 scat_sem, add=True)
                return c3
            lax.fori_loop(0, _ROWS, _scat, 0)
            phase += 1

    pltpu.make_async_copy(px_hbm.at[pl.ds(0, _ROWS * 128)],
                          zb.at[pl.ds(0, _ROWS * 128)], scat_sem).wait()
    plsc.subcore_barrier()

    pltpu.sync_copy(grid.at[pl.ds(tile_base, _TILE_WORDS)],
                    out_hbm.at[pl.ds(ci * _HALF_WORDS + tile_base,
                                     _TILE_WORDS)])


def _huber_body(g_ref, o_ref):
    d = g_ref[...]
    ad = jnp.abs(d)
    o_ref[0, 0] = jnp.sum(jnp.where(ad < 1.0, 0.5 * d * d, ad - 0.5))


@jax.jit
def _divroc_sc(px, py, pz, gx, gy, gz, cx, cy, cz):
    mesh = plsc.VectorSubcoreMesh(
        core_axis_name="c", subcore_axis_name="s",
        num_cores=_NC, num_subcores=_NS)
    fn = pl.kernel(
        _sc_body,
        out_type=jax.ShapeDtypeStruct((_NC * _HALF_WORDS,), jnp.float32),
        mesh=mesh,
        scratch_types=[
            pltpu.VMEM((_CHUNK,), jnp.float32),
            pltpu.VMEM((_CHUNK,), jnp.float32),
            pltpu.VMEM((_CHUNK,), jnp.float32),
            pltpu.VMEM((_CHUNK,), jnp.float32),
            pltpu.VMEM((_CHUNK,), jnp.float32),
            pltpu.VMEM((_CHUNK,), jnp.float32),
            pltpu.VMEM((_ROWS, 128), jnp.int32),
            pltpu.VMEM((_ROWS, 128), jnp.float32),
            pltpu.VMEM((_ROWS, 128), jnp.int32),
            pltpu.VMEM((_ROWS, 128), jnp.float32),
            pltpu.VMEM((_ZCHUNK,), jnp.float32),
            pltpu.VMEM_SHARED((_HALF_WORDS,), jnp.float32),
            pltpu.SemaphoreType.DMA,
            pltpu.SemaphoreType.DMA,
        ],
    )
    zeros_seed = jnp.zeros((_HALF_WORDS,), jnp.float32)
    diff_grid = fn(px, py, pz, gx, gy, gz, cx, cy, cz, zeros_seed)
    hsum = pl.pallas_call(
        _huber_body,
        out_shape=jax.ShapeDtypeStruct((1, 1), jnp.float32),
        out_specs=pl.BlockSpec(memory_space=pltpu.SMEM),
    )(diff_grid.reshape(1024, 2048))
    return hsum[0, 0]


def kernel(registration_pred, registration_gt, coords, wandb):
    n = registration_pred.shape[1]
    p = registration_pred.reshape(n, 3)
    g = registration_gt.reshape(n, 3)
    c = coords.reshape(n, 3)
    return _divroc_sc(p[:, 0], p[:, 1], p[:, 2],
                      g[:, 0], g[:, 1], g[:, 2],
                      c[:, 0], c[:, 1], c[:, 2])


# speedup vs baseline: 2.7572x; 1.5860x over previous
import jax
import jax.numpy as jnp
from jax import lax
from jax.experimental import pallas as pl
from jax.experimental.pallas import tpu as pltpu
from jax.experimental.pallas import tpu_sc as plsc

_N = 131072
_D = _H = _W = 128
_NC = 2
_NS = 16
_CHUNK = 2048
_PER_TILE = _N // _NS
_SUBCHUNKS = _PER_TILE // _CHUNK
_HALF_WORDS = (_D // _NC) * _H * _W
_TILE_WORDS = _HALF_WORDS // _NS
_GROUPS = _CHUNK // 16
_ROWS = _CHUNK // 32
_ZCHUNK = 16384
_SENT = _HALF_WORDS


def _floor_i32(f):
    t = f.astype(jnp.int32)
    tf = t.astype(jnp.float32)
    return jnp.where(tf > f, t - 1, t)


def _axis_terms(coord, lim):
    f = ((coord + 1.0) * 128.0 - 1.0) * 0.5
    c0 = _floor_i32(f)
    t = f - c0.astype(jnp.float32)
    v0 = (c0 >= 0) & (c0 <= lim - 1)
    v1 = (c0 >= -1) & (c0 <= lim - 2)
    a0 = jnp.where(v0, 1.0 - t, 0.0)
    a1 = jnp.where(v1, t, 0.0)
    i0 = jnp.clip(c0, 0, lim - 1)
    i1 = jnp.clip(c0 + 1, 0, lim - 1)
    return a0, a1, i0, i1


def _sc_body(px_hbm, py_hbm, pz_hbm, gx_hbm, gy_hbm, gz_hbm,
             cx_hbm, cy_hbm, cz_hbm, zeros_hbm, out_hbm,
             pxb, pyb, pzb, cxb, cyb, czb,
             idx_st0, val_st0, idx_st1, val_st1, zb, grid,
             in_sem, scat_sem):
    pb = (pxb, pyb, pzb)
    cb = (cxb, cyb, czb)
    pred_hbm = (px_hbm, py_hbm, pz_hbm)
    gt_hbm = (gx_hbm, gy_hbm, gz_hbm)
    coords_hbm = (cx_hbm, cy_hbm, cz_hbm)
    stages = ((idx_st0, val_st0), (idx_st1, val_st1))
    ci = lax.axis_index("c")
    si = lax.axis_index("s")
    tile_base = si * _TILE_WORDS

    pltpu.sync_copy(zeros_hbm.at[pl.ds(si * _TILE_WORDS, _TILE_WORDS)],
                    grid.at[pl.ds(tile_base, _TILE_WORDS)])
    plsc.subcore_barrier()

    phase = 0
    for src_hbm, sign in ((pred_hbm, 1.0), (gt_hbm, -1.0)):
        for sub in range(_SUBCHUNKS):
            ist, vst = stages[phase % 2]
            base = si * _PER_TILE + sub * _CHUNK
            for d in range(3):
                pltpu.async_copy(src_hbm[d].at[pl.ds(base, _CHUNK)], pb[d],
                                 in_sem)
                pltpu.async_copy(coords_hbm[d].at[pl.ds(base, _CHUNK)], cb[d],
                                 in_sem)
            pltpu.make_async_copy(px_hbm.at[pl.ds(0, 6 * _CHUNK)],
                                  zb.at[pl.ds(0, 6 * _CHUNK)], in_sem).wait()

            def _group(i, c2, ist=ist, vst=vst, sign=sign):
                s16 = pl.ds(i * 16, 16)
                x = pb[0][s16] + cb[0][s16]
                y = pb[1][s16] + cb[1][s16]
                z = pb[2][s16] + cb[2][s16]
                ax0, ax1, xi0, xi1 = _axis_terms(x, _W)
                ay0, ay1, yi0, yi1 = _axis_terms(y, _H)
                fz = ((z + 1.0) * 128.0 - 1.0) * 0.5
                z0 = _floor_i32(fz)
                tz = fz - z0.astype(jnp.float32)
                dz = (z0 & 1) ^ ci
                zc = z0 + dz
                wz = jnp.where(dz == 0, 1.0 - tz, tz)
                vzc = (zc >= 0) & (zc <= _D - 1)
                azc = jnp.where(vzc, wz * sign, 0.0)
                zbase = (jnp.clip(zc, 0, _D - 1) >> 1) * (_H * _W)

                b0 = zbase + yi0 * _W
                b1 = zbase + yi1 * _W
                a0 = azc * ay0
                a1 = azc * ay1
                row = i >> 1
                colb = (i & 1) * 64
                for k2, (bv, iv, av, xv) in enumerate(
                        ((b0, xi0, a0, ax0), (b0, xi1, a0, ax1),
                         (b1, xi0, a1, ax0), (b1, xi1, a1, ax1))):
                    v = av * xv
                    iw = jnp.where(v == 0.0, _SENT, bv + iv)
                    ist[row, pl.ds(colb + k2 * 16, 16)] = iw
                    vst[row, pl.ds(colb + k2 * 16, 16)] = v
                return c2
            lax.fori_loop(0, _GROUPS, _group, 0)

            if phase > 0:
                pltpu.make_async_copy(px_hbm.at[pl.ds(0, _ROWS * 128)],
                                      zb.at[pl.ds(0, _ROWS * 128)],
                                      scat_sem).wait()

            def _scat(j, c3, ist=ist, vst=vst):
                pltpu.async_copy(
                    vst.at[j],
                    grid.at[plsc.Indices(ist.at[j], ignored_value=_SENT)],
                    scat_sem, add=True)
                return c3
            lax.fori_loop(0, _ROWS, _scat, 0)
            phase += 1

    pltpu.make_async_copy(px_hbm.at[pl.ds(0, _ROWS * 128)],
                          zb.at[pl.ds(0, _ROWS * 128)], scat_sem).wait()
    plsc.subcore_barrier()

    pltpu.sync_copy(grid.at[pl.ds(tile_base, _TILE_WORDS)],
                    out_hbm.at[pl.ds(ci * _HALF_WORDS + tile_base,
                                     _TILE_WORDS)])


def _huber_body(g_ref, o_ref):
    d = g_ref[...]
    ad = jnp.abs(d)
    o_ref[0, 0] = jnp.sum(jnp.where(ad < 1.0, 0.5 * d * d, ad - 0.5))


@jax.jit
def _divroc_sc(px, py, pz, gx, gy, gz, cx, cy, cz):
    mesh = plsc.VectorSubcoreMesh(
        core_axis_name="c", subcore_axis_name="s",
        num_cores=_NC, num_subcores=_NS)
    fn = pl.kernel(
        _sc_body,
        out_type=jax.ShapeDtypeStruct((_NC * _HALF_WORDS,), jnp.float32),
        mesh=mesh,
        scratch_types=[
            pltpu.VMEM((_CHUNK,), jnp.float32),
            pltpu.VMEM((_CHUNK,), jnp.float32),
            pltpu.VMEM((_CHUNK,), jnp.float32),
            pltpu.VMEM((_CHUNK,), jnp.float32),
            pltpu.VMEM((_CHUNK,), jnp.float32),
            pltpu.VMEM((_CHUNK,), jnp.float32),
            pltpu.VMEM((_ROWS, 128), jnp.int32),
            pltpu.VMEM((_ROWS, 128), jnp.float32),
            pltpu.VMEM((_ROWS, 128), jnp.int32),
            pltpu.VMEM((_ROWS, 128), jnp.float32),
            pltpu.VMEM((_ZCHUNK,), jnp.float32),
            pltpu.VMEM_SHARED((_HALF_WORDS + 8,), jnp.float32),
            pltpu.SemaphoreType.DMA,
            pltpu.SemaphoreType.DMA,
        ],
    )
    zeros_seed = jnp.zeros((_HALF_WORDS,), jnp.float32)
    diff_grid = fn(px, py, pz, gx, gy, gz, cx, cy, cz, zeros_seed)
    hsum = pl.pallas_call(
        _huber_body,
        out_shape=jax.ShapeDtypeStruct((1, 1), jnp.float32),
        out_specs=pl.BlockSpec(memory_space=pltpu.SMEM),
    )(diff_grid.reshape(1024, 2048))
    return hsum[0, 0]


def kernel(registration_pred, registration_gt, coords, wandb):
    n = registration_pred.shape[1]
    p = registration_pred.reshape(n, 3)
    g = registration_gt.reshape(n, 3)
    c = coords.reshape(n, 3)
    return _divroc_sc(p[:, 0], p[:, 1], p[:, 2],
                      g[:, 0], g[:, 1], g[:, 2],
                      c[:, 0], c[:, 1], c[:, 2])


# speedup vs baseline: 2.7776x; 1.0074x over previous
import jax
import jax.numpy as jnp
from jax import lax
from jax.experimental import pallas as pl
from jax.experimental.pallas import tpu as pltpu
from jax.experimental.pallas import tpu_sc as plsc

_N = 131072
_D = _H = _W = 128
_NC = 2
_NS = 16
_CHUNK = 2048
_PER_TILE = _N // _NS
_SUBCHUNKS = _PER_TILE // _CHUNK
_HALF_WORDS = (_D // _NC) * _H * _W
_TILE_WORDS = _HALF_WORDS // _NS
_GROUPS = _CHUNK // 16
_ROWS = _CHUNK // 32
_ZCHUNK = 16384
_SENT = _HALF_WORDS


def _floor_i32(f):
    t = f.astype(jnp.int32)
    tf = t.astype(jnp.float32)
    return jnp.where(tf > f, t - 1, t)


def _axis_terms(coord, lim):
    f = ((coord + 1.0) * 128.0 - 1.0) * 0.5
    c0 = _floor_i32(f)
    t = f - c0.astype(jnp.float32)
    v0 = (c0 >= 0) & (c0 <= lim - 1)
    v1 = (c0 >= -1) & (c0 <= lim - 2)
    a0 = jnp.where(v0, 1.0 - t, 0.0)
    a1 = jnp.where(v1, t, 0.0)
    return a0, a1, c0, c0 + 1


def _sc_body(px_hbm, py_hbm, pz_hbm, gx_hbm, gy_hbm, gz_hbm,
             cx_hbm, cy_hbm, cz_hbm, zeros_hbm, out_hbm,
             pxb, pyb, pzb, cxb, cyb, czb,
             idx_st0, val_st0, idx_st1, val_st1, zb, grid,
             in_sem, scat_sem):
    pb = (pxb, pyb, pzb)
    cb = (cxb, cyb, czb)
    pred_hbm = (px_hbm, py_hbm, pz_hbm)
    gt_hbm = (gx_hbm, gy_hbm, gz_hbm)
    coords_hbm = (cx_hbm, cy_hbm, cz_hbm)
    stages = ((idx_st0, val_st0), (idx_st1, val_st1))
    ci = lax.axis_index("c")
    si = lax.axis_index("s")
    tile_base = si * _TILE_WORDS

    pltpu.sync_copy(zeros_hbm.at[pl.ds(si * _TILE_WORDS, _TILE_WORDS)],
                    grid.at[pl.ds(tile_base, _TILE_WORDS)])
    plsc.subcore_barrier()

    phase = 0
    for src_hbm, sign in ((pred_hbm, 1.0), (gt_hbm, -1.0)):
        for sub in range(_SUBCHUNKS):
            ist, vst = stages[phase % 2]
            base = si * _PER_TILE + sub * _CHUNK
            for d in range(3):
                pltpu.async_copy(src_hbm[d].at[pl.ds(base, _CHUNK)], pb[d],
                                 in_sem)
                pltpu.async_copy(coords_hbm[d].at[pl.ds(base, _CHUNK)], cb[d],
                                 in_sem)
            pltpu.make_async_copy(px_hbm.at[pl.ds(0, 6 * _CHUNK)],
                                  zb.at[pl.ds(0, 6 * _CHUNK)], in_sem).wait()

            def _group(i, c2, ist=ist, vst=vst, sign=sign):
                s16 = pl.ds(i * 16, 16)
                x = pb[0][s16] + cb[0][s16]
                y = pb[1][s16] + cb[1][s16]
                z = pb[2][s16] + cb[2][s16]
                ax0, ax1, xi0, xi1 = _axis_terms(x, _W)
                ay0, ay1, yi0, yi1 = _axis_terms(y, _H)
                fz = ((z + 1.0) * 128.0 - 1.0) * 0.5
                z0 = _floor_i32(fz)
                tz = fz - z0.astype(jnp.float32)
                dz = (z0 & 1) ^ ci
                zc = z0 + dz
                wz = jnp.where(dz == 0, 1.0 - tz, tz)
                vzc = (zc >= 0) & (zc <= _D - 1)
                azc = jnp.where(vzc, wz * sign, 0.0)
                zbase = (zc >> 1) * (_H * _W)

                b0 = zbase + yi0 * _W
                b1 = zbase + yi1 * _W
                a0 = azc * ay0
                a1 = azc * ay1
                row = i >> 1
                colb = (i & 1) * 64
                for k2, (bv, iv, av, xv) in enumerate(
                        ((b0, xi0, a0, ax0), (b0, xi1, a0, ax1),
                         (b1, xi0, a1, ax0), (b1, xi1, a1, ax1))):
                    v = av * xv
                    iw = jnp.where(v == 0.0, _SENT, bv + iv)
                    ist[row, pl.ds(colb + k2 * 16, 16)] = iw
                    vst[row, pl.ds(colb + k2 * 16, 16)] = v
                return c2
            lax.fori_loop(0, _GROUPS, _group, 0, unroll=2)

            if phase > 0:
                pltpu.make_async_copy(px_hbm.at[pl.ds(0, _ROWS * 128)],
                                      zb.at[pl.ds(0, _ROWS * 128)],
                                      scat_sem).wait()

            def _scat(j, c3, ist=ist, vst=vst):
                pltpu.async_copy(
                    vst.at[j],
                    grid.at[plsc.Indices(ist.at[j], ignored_value=_SENT)],
                    scat_sem, add=True)
                return c3
            lax.fori_loop(0, _ROWS, _scat, 0)
            phase += 1

    pltpu.make_async_copy(px_hbm.at[pl.ds(0, _ROWS * 128)],
                          zb.at[pl.ds(0, _ROWS * 128)], scat_sem).wait()
    plsc.subcore_barrier()

    pltpu.sync_copy(grid.at[pl.ds(tile_base, _TILE_WORDS)],
                    out_hbm.at[pl.ds(ci * _HALF_WORDS + tile_base,
                                     _TILE_WORDS)])


def _huber_body(g_ref, o_ref):
    d = g_ref[...]
    ad = jnp.abs(d)
    o_ref[0, 0] = jnp.sum(jnp.where(ad < 1.0, 0.5 * d * d, ad - 0.5))


@jax.jit
def _divroc_sc(px, py, pz, gx, gy, gz, cx, cy, cz):
    mesh = plsc.VectorSubcoreMesh(
        core_axis_name="c", subcore_axis_name="s",
        num_cores=_NC, num_subcores=_NS)
    fn = pl.kernel(
        _sc_body,
        out_type=jax.ShapeDtypeStruct((_NC * _HALF_WORDS,), jnp.float32),
        mesh=mesh,
        scratch_types=[
            pltpu.VMEM((_CHUNK,), jnp.float32),
            pltpu.VMEM((_CHUNK,), jnp.float32),
            pltpu.VMEM((_CHUNK,), jnp.float32),
            pltpu.VMEM((_CHUNK,), jnp.float32),
            pltpu.VMEM((_CHUNK,), jnp.float32),
            pltpu.VMEM((_CHUNK,), jnp.float32),
            pltpu.VMEM((_ROWS, 128), jnp.int32),
            pltpu.VMEM((_ROWS, 128), jnp.float32),
            pltpu.VMEM((_ROWS, 128), jnp.int32),
            pltpu.VMEM((_ROWS, 128), jnp.float32),
            pltpu.VMEM((_ZCHUNK,), jnp.float32),
            pltpu.VMEM_SHARED((_HALF_WORDS + 8,), jnp.float32),
            pltpu.SemaphoreType.DMA,
            pltpu.SemaphoreType.DMA,
        ],
    )
    zeros_seed = jnp.zeros((_HALF_WORDS,), jnp.float32)
    diff_grid = fn(px, py, pz, gx, gy, gz, cx, cy, cz, zeros_seed)
    hsum = pl.pallas_call(
        _huber_body,
        out_shape=jax.ShapeDtypeStruct((1, 1), jnp.float32),
        out_specs=pl.BlockSpec(memory_space=pltpu.SMEM),
    )(diff_grid.reshape(1024, 2048))
    return hsum[0, 0]


def kernel(registration_pred, registration_gt, coords, wandb):
    n = registration_pred.shape[1]
    p = registration_pred.reshape(n, 3)
    g = registration_gt.reshape(n, 3)
    c = coords.reshape(n, 3)
    return _divroc_sc(p[:, 0], p[:, 1], p[:, 2],
                      g[:, 0], g[:, 1], g[:, 2],
                      c[:, 0], c[:, 1], c[:, 2])


# speedup vs baseline: 2.8483x; 1.0255x over previous
import jax
import jax.numpy as jnp
from jax import lax
from jax.experimental import pallas as pl
from jax.experimental.pallas import tpu as pltpu
from jax.experimental.pallas import tpu_sc as plsc

_N = 131072
_D = _H = _W = 128
_NC = 2
_NS = 16
_CHUNK = 2048
_PER_TILE = _N // _NS
_SUBCHUNKS = _PER_TILE // _CHUNK
_HALF_WORDS = (_D // _NC) * _H * _W
_TILE_WORDS = _HALF_WORDS // _NS
_GROUPS = _CHUNK // 16
_ROWS = _CHUNK // 32
_ZCHUNK = 16384
_SENT = _HALF_WORDS


def _floor_i32(f):
    t = f.astype(jnp.int32)
    tf = t.astype(jnp.float32)
    return jnp.where(tf > f, t - 1, t)


def _axis_terms(coord, lim):
    f = ((coord + 1.0) * 128.0 - 1.0) * 0.5
    c0 = _floor_i32(f)
    t = f - c0.astype(jnp.float32)
    v0 = (c0 >= 0) & (c0 <= lim - 1)
    v1 = (c0 >= -1) & (c0 <= lim - 2)
    a0 = jnp.where(v0, 1.0 - t, 0.0)
    a1 = jnp.where(v1, t, 0.0)
    return a0, a1, c0, c0 + 1


def _sc_body(pred_hbm, gt_hbm, coords_hbm, zeros_hbm, out_hbm,
             pxb, pyb, pzb, cxb, cyb, czb,
             idx_st0, val_st0, idx_st1, val_st1, zb, accv, grid,
             in_sem, scat_sem):
    pb = (pxb, pyb, pzb)
    cb = (cxb, cyb, czb)
    stages = ((idx_st0, val_st0), (idx_st1, val_st1))
    ci = lax.axis_index("c")
    si = lax.axis_index("s")
    tile_base = si * _TILE_WORDS

    pltpu.sync_copy(zeros_hbm.at[pl.ds(si * _TILE_WORDS, _TILE_WORDS)],
                    grid.at[pl.ds(tile_base, _TILE_WORDS)])
    plsc.subcore_barrier()

    phase = 0
    for src_hbm, sign in ((pred_hbm, 1.0), (gt_hbm, -1.0)):
        for sub in range(_SUBCHUNKS):
            ist, vst = stages[phase % 2]
            base = si * _PER_TILE + sub * _CHUNK
            for d in range(3):
                pltpu.async_copy(src_hbm.at[pl.ds(d * _N + base, _CHUNK)],
                                 pb[d], in_sem)
                pltpu.async_copy(coords_hbm.at[pl.ds(d * _N + base, _CHUNK)],
                                 cb[d], in_sem)
            pltpu.make_async_copy(pred_hbm.at[pl.ds(0, 6 * _CHUNK)],
                                  zb.at[pl.ds(0, 6 * _CHUNK)], in_sem).wait()

            def _group(i, c2, ist=ist, vst=vst, sign=sign):
                s16 = pl.ds(i * 16, 16)
                x = pb[0][s16] + cb[0][s16]
                y = pb[1][s16] + cb[1][s16]
                z = pb[2][s16] + cb[2][s16]
                ax0, ax1, xi0, xi1 = _axis_terms(x, _W)
                ay0, ay1, yi0, yi1 = _axis_terms(y, _H)
                fz = ((z + 1.0) * 128.0 - 1.0) * 0.5
                z0 = _floor_i32(fz)
                tz = fz - z0.astype(jnp.float32)
                dz = (z0 & 1) ^ ci
                zc = z0 + dz
                wz = jnp.where(dz == 0, 1.0 - tz, tz)
                vzc = (zc >= 0) & (zc <= _D - 1)
                azc = jnp.where(vzc, wz * sign, 0.0)
                zbase = (zc >> 1) * (_H * _W)

                b0 = zbase + yi0 * _W
                b1 = zbase + yi1 * _W
                a0 = azc * ay0
                a1 = azc * ay1
                row = i >> 1
                colb = (i & 1) * 64
                for k2, (bv, iv, av, xv) in enumerate(
                        ((b0, xi0, a0, ax0), (b0, xi1, a0, ax1),
                         (b1, xi0, a1, ax0), (b1, xi1, a1, ax1))):
                    v = av * xv
                    iw = jnp.where(v == 0.0, _SENT, bv + iv)
                    ist[row, pl.ds(colb + k2 * 16, 16)] = iw
                    vst[row, pl.ds(colb + k2 * 16, 16)] = v
                return c2
            lax.fori_loop(0, _GROUPS, _group, 0, unroll=2)

            if phase > 0:
                pltpu.make_async_copy(pred_hbm.at[pl.ds(0, _ROWS * 128)],
                                      zb.at[pl.ds(0, _ROWS * 128)],
                                      scat_sem).wait()

            def _scat(j, c3, ist=ist, vst=vst):
                pltpu.async_copy(
                    vst.at[j],
                    grid.at[plsc.Indices(ist.at[j], ignored_value=_SENT)],
                    scat_sem, add=True)
                return c3
            lax.fori_loop(0, _ROWS, _scat, 0)
            phase += 1

    pltpu.make_async_copy(pred_hbm.at[pl.ds(0, _ROWS * 128)],
                          zb.at[pl.ds(0, _ROWS * 128)], scat_sem).wait()
    plsc.subcore_barrier()

    zeros16 = jnp.zeros((16,), jnp.float32)

    def _hchunk(k, acc):
        pltpu.sync_copy(grid.at[pl.ds(tile_base + k * _ZCHUNK, _ZCHUNK)], zb)

        def _hstep(t, a):
            dv = zb[pl.ds(t * 16, 16)]
            ad = jnp.abs(dv)
            return a + jnp.where(ad < 1.0, 0.5 * dv * dv, ad - 0.5)
        return lax.fori_loop(0, _ZCHUNK // 16, _hstep, acc)
    acc = lax.fori_loop(0, _TILE_WORDS // _ZCHUNK, _hchunk, zeros16)
    accv[...] = acc
    pltpu.sync_copy(accv, out_hbm.at[pl.ds((ci * _NS + si) * 16, 16)])


@jax.jit
def _divroc_sc(pred_t, gt_t, coords_t):
    mesh = plsc.VectorSubcoreMesh(
        core_axis_name="c", subcore_axis_name="s",
        num_cores=_NC, num_subcores=_NS)
    fn = pl.kernel(
        _sc_body,
        out_type=jax.ShapeDtypeStruct((_NC * _NS * 16,), jnp.float32),
        mesh=mesh,
        scratch_types=[
            pltpu.VMEM((_CHUNK,), jnp.float32),
            pltpu.VMEM((_CHUNK,), jnp.float32),
            pltpu.VMEM((_CHUNK,), jnp.float32),
            pltpu.VMEM((_CHUNK,), jnp.float32),
            pltpu.VMEM((_CHUNK,), jnp.float32),
            pltpu.VMEM((_CHUNK,), jnp.float32),
            pltpu.VMEM((_ROWS, 128), jnp.int32),
            pltpu.VMEM((_ROWS, 128), jnp.float32),
            pltpu.VMEM((_ROWS, 128), jnp.int32),
            pltpu.VMEM((_ROWS, 128), jnp.float32),
            pltpu.VMEM((_ZCHUNK,), jnp.float32),
            pltpu.VMEM((16,), jnp.float32),
            pltpu.VMEM_SHARED((_HALF_WORDS + 8,), jnp.float32),
            pltpu.SemaphoreType.DMA,
            pltpu.SemaphoreType.DMA,
        ],
    )
    zeros_seed = jnp.zeros((_HALF_WORDS,), jnp.float32)
    parts = fn(pred_t, gt_t, coords_t, zeros_seed)
    return jnp.sum(parts)


def kernel(registration_pred, registration_gt, coords, wandb):
    n = registration_pred.shape[1]
    return _divroc_sc(registration_pred.reshape(n, 3).T.reshape(-1),
                      registration_gt.reshape(n, 3).T.reshape(-1),
                      coords.reshape(n, 3).T.reshape(-1))
